# Initial kernel scaffold; baseline (speedup 1.0000x reference)
import jax
import jax.numpy as jnp
from jax.experimental import pallas as pl


def kernel(x, adj, ea, wcat1, sp1, wcat2, sp2):
    raise NotImplementedError("write your pallas kernel here")


# speedup vs baseline: 2.1919x; 2.1919x over previous
import jax
import jax.numpy as jnp
from jax.experimental import pallas as pl
from jax.experimental.pallas import tpu as pltpu

SEG = 128
H = 2
C = 64
HC = H * C
TI = 64
JC = 128
SLOPE = 0.2


def _proj_body(x_ref, wcat_ref, sp_ref, xl_ref, td_ref):
    t = jnp.dot(x_ref[...], wcat_ref[...],
                preferred_element_type=jnp.float32) + sp_ref[0:1, :]
    xl_ref[...] = t[:, 0:HC]
    td_ref[...] = t[:, SEG:3 * SEG]


def _proj(x, wcat, sp):
    n = x.shape[0]
    fin = x.shape[1]
    return pl.pallas_call(
        _proj_body,
        out_shape=(jax.ShapeDtypeStruct((n, HC), jnp.float32),
                   jax.ShapeDtypeStruct((n, 2 * SEG), jnp.float32)),
        grid=(2,),
        in_specs=[
            pl.BlockSpec((n // 2, fin), lambda i: (i, 0)),
            pl.BlockSpec((fin, 3 * SEG), lambda i: (0, 0)),
            pl.BlockSpec((8, 3 * SEG), lambda i: (0, 0)),
        ],
        out_specs=(pl.BlockSpec((n // 2, HC), lambda i: (i, 0)),
                   pl.BlockSpec((n // 2, 2 * SEG), lambda i: (i, 0))),
        compiler_params=pltpu.CompilerParams(
            dimension_semantics=("parallel",)),
    )(x, wcat, sp)


def _make_attn_body(n, apply_prelu):
    def body(xl_ref, td_ref, adj_ref, ea_ref, sp_ref, out_ref, s_ref):
        sp = sp_ref[...]
        we = sp[1:4, 0:HC]
        attf = sp[4:5, 0:HC]
        gb = sp[5:6, 0:C]
        pw = sp[6:7, 0:C]
        td = td_ref[...]
        xr = td[:, 0:HC]
        skip = td[:, SEG:SEG + C]
        adj = adj_ref[...]

        def chunk(jc, carry):
            j0 = jc * JC
            xl_c = xl_ref[pl.ds(j0, JC), :]
            ea0 = ea_ref[0, :, pl.ds(j0, JC)]
            ea1 = ea_ref[1, :, pl.ds(j0, JC)]
            ea2 = ea_ref[2, :, pl.ds(j0, JC)]
            m = (xr[:, None, :] + xl_c[None, :, :]
                 + ea0[:, :, None] * we[0:1, :]
                 + ea1[:, :, None] * we[1:2, :]
                 + ea2[:, :, None] * we[2:3, :])
            m = jnp.maximum(m, SLOPE * m)
            w = m * attf
            s_ref[0, :, pl.ds(j0, JC)] = jnp.sum(w[:, :, 0:C], axis=-1)
            s_ref[1, :, pl.ds(j0, JC)] = jnp.sum(w[:, :, C:HC], axis=-1)
            return carry

        jax.lax.fori_loop(0, n // JC, chunk, 0)

        bias = jnp.where(adj > 0, 0.0, -1e30)
        acc = None
        for h in range(H):
            s = s_ref[h] + bias
            smax = jnp.max(s, axis=1, keepdims=True)
            p = jnp.exp(s - smax) * adj
            denom = jnp.sum(p, axis=1, keepdims=True) + 1e-16
            alpha = p * pl.reciprocal(denom, approx=True)
            xl_h = xl_ref[:, h * C:(h + 1) * C]
            d = jnp.dot(alpha, xl_h, preferred_element_type=jnp.float32)
            acc = d if acc is None else acc + d
        out = acc * (1.0 / H) + gb + skip
        if apply_prelu:
            out = jnp.where(out > 0, out, pw * out)
        out_ref[...] = out
    return body


def _attn(xl, td, adj, ea, sp, apply_prelu):
    n = adj.shape[0]
    return pl.pallas_call(
        _make_attn_body(n, apply_prelu),
        out_shape=jax.ShapeDtypeStruct((n, C), jnp.float32),
        grid=(n // TI,),
        in_specs=[
            pl.BlockSpec((n, HC), lambda i: (0, 0)),
            pl.BlockSpec((TI, 2 * SEG), lambda i: (i, 0)),
            pl.BlockSpec((TI, n), lambda i: (i, 0)),
            pl.BlockSpec((3, TI, n), lambda i: (0, i, 0)),
            pl.BlockSpec((8, 3 * SEG), lambda i: (0, 0)),
        ],
        out_specs=pl.BlockSpec((TI, C), lambda i: (i, 0)),
        scratch_shapes=[pltpu.VMEM((H, TI, n), jnp.float32)],
        compiler_params=pltpu.CompilerParams(
            dimension_semantics=("parallel",)),
    )(xl, td, adj, ea, sp)


def _layer(x, adj, ea, wcat, sp, apply_prelu):
    xl, td = _proj(x, wcat, sp)
    return _attn(xl, td, adj, ea, sp, apply_prelu)


def kernel(x, adj, ea, wcat1, sp1, wcat2, sp2):
    h1 = _layer(x, adj, ea, wcat1, sp1, False)
    return _layer(h1, adj, ea, wcat2, sp2, True)


# speedup vs baseline: 9.1425x; 4.1710x over previous
import jax
import jax.numpy as jnp
from jax.experimental import pallas as pl
from jax.experimental.pallas import tpu as pltpu

SEG = 128
H = 2
C = 64
HC = H * C
TI = 64
JB = 256
SLOPE = 0.2


def _proj_body(x_ref, wcat_ref, sp_ref, xlt_ref, td_ref):
    t = jnp.dot(x_ref[...], wcat_ref[...],
                preferred_element_type=jnp.float32) + sp_ref[0:1, :]
    xlt_ref[...] = t[:, 0:HC].T
    td_ref[...] = t[:, SEG:3 * SEG]


def _proj(x, wcat, sp):
    n = x.shape[0]
    fin = x.shape[1]
    return pl.pallas_call(
        _proj_body,
        out_shape=(jax.ShapeDtypeStruct((HC, n), jnp.float32),
                   jax.ShapeDtypeStruct((n, 2 * SEG), jnp.float32)),
        grid=(2,),
        in_specs=[
            pl.BlockSpec((n // 2, fin), lambda i: (i, 0)),
            pl.BlockSpec((fin, 3 * SEG), lambda i: (0, 0)),
            pl.BlockSpec((8, 3 * SEG), lambda i: (0, 0)),
        ],
        out_specs=(pl.BlockSpec((HC, n // 2), lambda i: (0, i)),
                   pl.BlockSpec((n // 2, 2 * SEG), lambda i: (i, 0))),
        compiler_params=pltpu.CompilerParams(
            dimension_semantics=("parallel",)),
    )(x, wcat, sp)


def _make_attn_body(n, apply_prelu):
    def body(xlt_ref, td_ref, adj_ref, ea_ref, sp_ref, sps_ref, out_ref,
             s_ref):
        td = td_ref[...]
        xr = td[:, 0:HC]
        skip = td[:, SEG:SEG + C]
        lane_iota = jax.lax.broadcasted_iota(jnp.int32, (TI, HC), 1)

        s_ref[...] = jnp.zeros((H, TI, n), jnp.float32)

        def make_octet(h):
            def octet(o, carry):
                c0 = h * C + o * 8
                xl8 = xlt_ref[pl.ds(c0, 8), :]
                for jb in range(n // JB):
                    j0 = jb * JB
                    acc = s_ref[h, :, j0:j0 + JB]
                    for u in range(8):
                        c = c0 + u
                        we0 = sps_ref[1, c]
                        we1 = sps_ref[2, c]
                        we2 = sps_ref[3, c]
                        af = sps_ref[4, c]
                        xr_col = jnp.sum(
                            jnp.where(lane_iota == c, xr, 0.0),
                            axis=1, keepdims=True)
                        v = (ea_ref[0, :, j0:j0 + JB] * we0
                             + ea_ref[1, :, j0:j0 + JB] * we1
                             + ea_ref[2, :, j0:j0 + JB] * we2
                             + xl8[u:u + 1, j0:j0 + JB]
                             + xr_col)
                        acc = acc + jnp.maximum(v, SLOPE * v) * af
                    s_ref[h, :, j0:j0 + JB] = acc
                return carry
            return octet

        for h in range(H):
            jax.lax.fori_loop(0, C // 8, make_octet(h), 0)

        adj = adj_ref[...]
        sp = sp_ref[...]
        gb = sp[5:6, 0:C]
        pw = sp[6:7, 0:C]
        bias = jnp.where(adj > 0, 0.0, -1e30)
        acc = None
        for h in range(H):
            s = s_ref[h] + bias
            smax = jnp.max(s, axis=1, keepdims=True)
            p = jnp.exp(s - smax) * adj
            denom = jnp.sum(p, axis=1, keepdims=True) + 1e-16
            alpha = p * pl.reciprocal(denom, approx=True)
            xlt_h = xlt_ref[h * C:(h + 1) * C, :]
            d = jnp.einsum('tn,cn->tc', alpha, xlt_h,
                           preferred_element_type=jnp.float32)
            acc = d if acc is None else acc + d
        out = acc * (1.0 / H) + gb + skip
        if apply_prelu:
            out = jnp.where(out > 0, out, pw * out)
        out_ref[...] = out
    return body


def _attn(xlt, td, adj, ea, sp, apply_prelu):
    n = adj.shape[0]
    return pl.pallas_call(
        _make_attn_body(n, apply_prelu),
        out_shape=jax.ShapeDtypeStruct((n, C), jnp.float32),
        grid=(n // TI,),
        in_specs=[
            pl.BlockSpec((HC, n), lambda i: (0, 0)),
            pl.BlockSpec((TI, 2 * SEG), lambda i: (i, 0)),
            pl.BlockSpec((TI, n), lambda i: (i, 0)),
            pl.BlockSpec((3, TI, n), lambda i: (0, i, 0)),
            pl.BlockSpec((8, 3 * SEG), lambda i: (0, 0)),
            pl.BlockSpec(memory_space=pltpu.MemorySpace.SMEM),
        ],
        out_specs=pl.BlockSpec((TI, C), lambda i: (i, 0)),
        scratch_shapes=[pltpu.VMEM((H, TI, n), jnp.float32)],
        compiler_params=pltpu.CompilerParams(
            dimension_semantics=("parallel",)),
    )(xlt, td, adj, ea, sp, sp)


def _layer(x, adj, ea, wcat, sp, apply_prelu):
    xlt, td = _proj(x, wcat, sp)
    return _attn(xlt, td, adj, ea, sp, apply_prelu)


def kernel(x, adj, ea, wcat1, sp1, wcat2, sp2):
    h1 = _layer(x, adj, ea, wcat1, sp1, False)
    return _layer(h1, adj, ea, wcat2, sp2, True)
